# Initial kernel scaffold; baseline (speedup 1.0000x reference)
import jax
import jax.numpy as jnp
from jax.experimental import pallas as pl


def kernel(variables, factors, senders, receivers, W_msg, b_msg, W_comb, b_comb):
    raise NotImplementedError("write your pallas kernel here")


# speedup vs baseline: 1.7677x; 1.7677x over previous
import functools

import jax
import jax.numpy as jnp
from jax import lax
from jax.experimental import pallas as pl
from jax.experimental.pallas import tpu as pltpu
from jax.experimental.pallas import tpu_sc as plsc

N_NODE = 10000
D = 256
E_TOTAL = 160000
NC, NS = 2, 16
NTILE = NC * NS

NP = 10240
EPT = 10240
E_PAD = NS * EPT
CHUNK = 128
BLK1 = 1024
BLK2 = 1000


def _project_body(x_ref, w_ref, b_ref, o_ref):
    cc = pl.program_id(0)
    jj = pl.program_id(1)
    is_fac = (jj < NP // BLK1).astype(jnp.float32)
    acc = jnp.dot(x_ref[...], w_ref[...], preferred_element_type=jnp.float32)
    o_ref[...] = acc + is_fac * b_ref[pl.ds(cc, 1), :]


def _project(x_pad, w3, b2):
    grid = (NC, 2 * NP // BLK1)
    return pl.pallas_call(
        _project_body,
        grid=grid,
        in_specs=[
            pl.BlockSpec((BLK1, D), lambda c, j: (j, 0)),
            pl.BlockSpec((256, 128), lambda c, j: (j // (NP // BLK1), c)),
            pl.BlockSpec((2, 128), lambda c, j: (0, 0)),
        ],
        out_specs=pl.BlockSpec((BLK1, 128), lambda c, j: (c * (2 * NP // BLK1) + j, 0)),
        out_shape=jax.ShapeDtypeStruct((2 * 2 * NP, 128), jnp.float32),
    )(x_pad, w3, b2)


def _edge_sc_body(ptab, snd, rcv, out, agg, rbuf, sbuf, gf, gv, pf, pv, sem1, sem2):
    c = lax.axis_index("c")
    s = lax.axis_index("s")

    def _zrow(i, _):
        for j in range(8):
            pf[i, pl.ds(j * 16, 16)] = jnp.zeros((16,), jnp.float32)
        return _
    lax.fori_loop(0, CHUNK, _zrow, None)
    for k in range(NP // (NS * CHUNK)):
        pltpu.sync_copy(pf, agg.at[pl.ds((s * (NP // (NS * CHUNK)) + k) * CHUNK, CHUNK)])
    plsc.subcore_barrier()

    offf = c * (2 * NP)
    offv = c * (2 * NP) + NP

    def _chunk(k, _):
        b = s * EPT + k * CHUNK
        pltpu.sync_copy(rcv.at[pl.ds(b, CHUNK)], rbuf)
        pltpu.sync_copy(snd.at[pl.ds(b, CHUNK)], sbuf)
        for j in range(CHUNK // 16):
            sl = pl.ds(j * 16, 16)
            gf[sl] = rbuf[sl] + offf
            gv[sl] = sbuf[sl] + offv
        cp1 = pltpu.async_copy(ptab.at[gf], pf, sem1)
        cp2 = pltpu.async_copy(ptab.at[gv], pv, sem2)
        cp1.wait()
        cp2.wait()

        def _row(i, _2):
            for j in range(8):
                sl = pl.ds(j * 16, 16)
                pf[i, sl] = jnp.maximum(pf[i, sl] + pv[i, sl], 0.0)
            return _2
        lax.fori_loop(0, CHUNK, _row, None, unroll=2)
        pltpu.sync_copy(pf, agg.at[rbuf], add=True)
        return _
    lax.fori_loop(0, EPT // CHUNK, _chunk, None)

    plsc.subcore_barrier()
    rows = NP // NS
    pltpu.sync_copy(agg.at[pl.ds(s * rows, rows)], out.at[c, pl.ds(s * rows, rows)])


@functools.cache
def _edge_sc():
    mesh = plsc.VectorSubcoreMesh(
        core_axis_name="c", subcore_axis_name="s", num_cores=NC, num_subcores=NS)
    return pl.kernel(
        _edge_sc_body,
        out_type=jax.ShapeDtypeStruct((NC, NP, 128), jnp.float32),
        mesh=mesh,
        scratch_types=[
            pltpu.VMEM_SHARED((NP, 128), jnp.float32),
            pltpu.VMEM((CHUNK,), jnp.int32),
            pltpu.VMEM((CHUNK,), jnp.int32),
            pltpu.VMEM((CHUNK,), jnp.int32),
            pltpu.VMEM((CHUNK,), jnp.int32),
            pltpu.VMEM((CHUNK, 128), jnp.float32),
            pltpu.VMEM((CHUNK, 128), jnp.float32),
            pltpu.SemaphoreType.DMA,
            pltpu.SemaphoreType.DMA,
        ],
    )


def _combine_body(f_ref, a_ref, w_ref, b_ref, o_ref):
    acc = jnp.dot(f_ref[...], w_ref[0:256, :], preferred_element_type=jnp.float32)
    acc += jnp.dot(a_ref[0], w_ref[256:384, :], preferred_element_type=jnp.float32)
    acc += jnp.dot(a_ref[1], w_ref[384:512, :], preferred_element_type=jnp.float32)
    o_ref[...] = jnp.maximum(acc + b_ref[...], 0.0)


def _combine(factors, agg, w_comb, b2):
    grid = (N_NODE // BLK2,)
    return pl.pallas_call(
        _combine_body,
        grid=grid,
        in_specs=[
            pl.BlockSpec((BLK2, D), lambda i: (i, 0)),
            pl.BlockSpec((NC, BLK2, 128), lambda i: (0, i, 0)),
            pl.BlockSpec((2 * D, D), lambda i: (0, 0)),
            pl.BlockSpec((1, D), lambda i: (0, 0)),
        ],
        out_specs=pl.BlockSpec((BLK2, D), lambda i: (i, 0)),
        out_shape=jax.ShapeDtypeStruct((N_NODE, D), jnp.float32),
    )(factors, agg, w_comb, b2)


def kernel(variables, factors, senders, receivers, W_msg, b_msg, W_comb, b_comb):
    x_pad = jnp.zeros((2 * NP, D), jnp.float32)
    x_pad = x_pad.at[0:N_NODE].set(factors)
    x_pad = x_pad.at[NP:NP + N_NODE].set(variables)
    ptab = _project(x_pad, W_msg, b_msg.reshape(2, 128))

    snd_pad = jnp.concatenate([senders, jnp.zeros((E_PAD - E_TOTAL,), jnp.int32)])
    rcv_pad = jnp.concatenate(
        [receivers, jnp.full((E_PAD - E_TOTAL,), N_NODE, jnp.int32)])

    agg = _edge_sc()(ptab, snd_pad, rcv_pad)

    out = _combine(factors, agg, W_comb, b_comb.reshape(1, D))
    return out


# speedup vs baseline: 4.4886x; 2.5393x over previous
import functools

import jax
import jax.numpy as jnp
from jax import lax
from jax.experimental import pallas as pl
from jax.experimental.pallas import tpu as pltpu
from jax.experimental.pallas import tpu_sc as plsc

N_NODE = 10000
D = 256
E_TOTAL = 160000
NC, NS = 2, 16

NP = 10240
NAGG = 10112
E_PAD = 163840
EPC = E_PAD // NC
EPT = EPC // NS
CHUNK = 32
BLK1 = 1024
BLK2 = 1000


def _project_body(x_ref, w_ref, b_ref, o_ref):
    jj = pl.program_id(0)
    is_fac = (jj < NP // BLK1).astype(jnp.float32)
    acc = jnp.dot(x_ref[...], w_ref[...], preferred_element_type=jnp.float32)
    o_ref[...] = (acc + is_fac * b_ref[...]).astype(jnp.bfloat16)


def _project(x_pad, w_msg, b2):
    grid = (2 * NP // BLK1,)
    return pl.pallas_call(
        _project_body,
        grid=grid,
        in_specs=[
            pl.BlockSpec((BLK1, D), lambda j: (j, 0)),
            pl.BlockSpec((D, D), lambda j: (j // (NP // BLK1), 0)),
            pl.BlockSpec((1, D), lambda j: (0, 0)),
        ],
        out_specs=pl.BlockSpec((BLK1, D), lambda j: (j, 0)),
        out_shape=jax.ShapeDtypeStruct((2 * NP, D), jnp.bfloat16),
    )(x_pad, w_msg, b2)


def _edge_sc_body(ptab, snd, rcv, out, agg, rall, sall,
                  gf0, gf1, gv0, gv1, rs0, rs1,
                  pf0, pf1, pv0, pv1, mg0, mg1,
                  sgf0, sgf1, sgv0, sgv1, ssc0, ssc1):
    c = lax.axis_index("c")
    s = lax.axis_index("s")
    GF, GV, RS = (gf0, gf1), (gv0, gv1), (rs0, rs1)
    PF, PV, MG = (pf0, pf1), (pv0, pv1), (mg0, mg1)
    SGF, SGV, SSC = (sgf0, sgf1), (sgv0, sgv1), (ssc0, ssc1)
    NCH = EPT // CHUNK
    TROWS = NAGG // NS

    pltpu.sync_copy(rcv.at[pl.ds(c * EPC + s * EPT, EPT)], rall)
    pltpu.sync_copy(snd.at[pl.ds(c * EPC + s * EPT, EPT)], sall)

    def _zrow(i, _):
        for j in range(8):
            mg0[i, pl.ds(j * 32, 32)] = jnp.zeros((32,), jnp.bfloat16)
        return _
    lax.fori_loop(0, CHUNK, _zrow, None)
    for k in range(TROWS // CHUNK):
        pltpu.sync_copy(mg0, agg.at[pl.ds(s * TROWS + k * CHUNK, CHUNK)])
    rem = TROWS % CHUNK
    if rem:
        pltpu.sync_copy(mg0.at[pl.ds(0, rem)],
                        agg.at[pl.ds(s * TROWS + (TROWS // CHUNK) * CHUNK, rem)])
    plsc.subcore_barrier()

    def prep_gidx(k, slot):
        base = k * CHUNK
        for j in range(CHUNK // 16):
            sl = pl.ds(j * 16, 16)
            GF[slot][sl] = rall[pl.ds(base + j * 16, 16)]
            GV[slot][sl] = sall[pl.ds(base + j * 16, 16)] + NP

    def issue_gather(k, slot):
        prep_gidx(k, slot)
        pltpu.async_copy(ptab.at[GF[slot]], PF[slot], SGF[slot])
        pltpu.async_copy(ptab.at[GV[slot]], PV[slot], SGV[slot])

    def wait_gather(slot):
        pltpu.make_async_copy(ptab.at[GF[slot]], PF[slot], SGF[slot]).wait()
        pltpu.make_async_copy(ptab.at[GV[slot]], PV[slot], SGV[slot]).wait()

    def compute(slot):
        def _row(i, _2):
            a = [PF[slot][i, pl.ds(j * 32, 32)] for j in range(8)]
            b = [PV[slot][i, pl.ds(j * 32, 32)] for j in range(8)]
            m = [jnp.maximum(a[j] + b[j], jnp.bfloat16(0.0)) for j in range(8)]
            for j in range(8):
                MG[slot][i, pl.ds(j * 32, 32)] = m[j]
            return _2
        lax.fori_loop(0, CHUNK, _row, None, unroll=2)

    def prep_sidx(k, slot):
        base = k * CHUNK
        for j in range(CHUNK // 16):
            RS[slot][pl.ds(j * 16, 16)] = rall[pl.ds(base + j * 16, 16)]

    def issue_scatter(slot):
        pltpu.async_copy(MG[slot], agg.at[RS[slot]], SSC[slot], add=True)

    def wait_scatter(slot):
        pltpu.make_async_copy(MG[slot], agg.at[RS[slot]], SSC[slot]).wait()

    issue_gather(0, 0)
    issue_gather(1, 1)
    for slot in (0, 1):
        wait_gather(slot)
        compute(slot)
        prep_sidx(slot, slot)
        issue_scatter(slot)
        issue_gather(slot + 2, slot)

    def _pair(p, _):
        for slot in (0, 1):
            k = 2 * p + slot
            wait_gather(slot)
            wait_scatter(slot)
            compute(slot)
            prep_sidx(k, slot)
            issue_scatter(slot)

            @pl.when(k + 2 < NCH)
            def _():
                issue_gather(k + 2, slot)
        return _
    lax.fori_loop(1, NCH // 2, _pair, None)
    wait_scatter(0)
    wait_scatter(1)
    plsc.subcore_barrier()

    pltpu.sync_copy(agg.at[pl.ds(s * TROWS, TROWS)],
                    out.at[c, pl.ds(s * TROWS, TROWS)])


@functools.cache
def _edge_sc():
    mesh = plsc.VectorSubcoreMesh(
        core_axis_name="c", subcore_axis_name="s", num_cores=NC, num_subcores=NS)
    return pl.kernel(
        _edge_sc_body,
        out_type=jax.ShapeDtypeStruct((NC, NAGG, D), jnp.bfloat16),
        mesh=mesh,
        scratch_types=(
            [pltpu.VMEM_SHARED((NAGG, D), jnp.bfloat16)]
            + [pltpu.VMEM((EPT,), jnp.int32)] * 2
            + [pltpu.VMEM((CHUNK,), jnp.int32)] * 6
            + [pltpu.VMEM((CHUNK, D), jnp.bfloat16)] * 6
            + [pltpu.SemaphoreType.DMA] * 6
        ),
        compiler_params=pltpu.CompilerParams(use_tc_tiling_on_sc=False),
    )


def _combine_body(f_ref, a_ref, w_ref, b_ref, o_ref):
    acc = jnp.dot(f_ref[...], w_ref[0:256, :], preferred_element_type=jnp.float32)
    agg = a_ref[0].astype(jnp.float32) + a_ref[1].astype(jnp.float32)
    acc += jnp.dot(agg, w_ref[256:512, :], preferred_element_type=jnp.float32)
    o_ref[...] = jnp.maximum(acc + b_ref[...], 0.0)


def _combine(factors, hagg, w_comb, b2):
    grid = (N_NODE // BLK2,)
    return pl.pallas_call(
        _combine_body,
        grid=grid,
        in_specs=[
            pl.BlockSpec((BLK2, D), lambda i: (i, 0)),
            pl.BlockSpec((NC, BLK2, D), lambda i: (0, i, 0)),
            pl.BlockSpec((2 * D, D), lambda i: (0, 0)),
            pl.BlockSpec((1, D), lambda i: (0, 0)),
        ],
        out_specs=pl.BlockSpec((BLK2, D), lambda i: (i, 0)),
        out_shape=jax.ShapeDtypeStruct((N_NODE, D), jnp.float32),
    )(factors, hagg, w_comb, b2)


def kernel(variables, factors, senders, receivers, W_msg, b_msg, W_comb, b_comb):
    x_pad = jnp.zeros((2 * NP, D), jnp.float32)
    x_pad = x_pad.at[0:N_NODE].set(factors)
    x_pad = x_pad.at[NP:NP + N_NODE].set(variables)

    ptab = _project(x_pad, W_msg, b_msg.reshape(1, D))

    snd_pad = jnp.concatenate([senders, jnp.zeros((E_PAD - E_TOTAL,), jnp.int32)])
    rcv_pad = jnp.concatenate(
        [receivers, jnp.full((E_PAD - E_TOTAL,), N_NODE, jnp.int32)])

    hagg = _edge_sc()(ptab, snd_pad, rcv_pad)

    out = _combine(factors, hagg, W_comb, b_comb.reshape(1, D))
    return out


# speedup vs baseline: 4.7775x; 1.0644x over previous
import functools

import jax
import jax.numpy as jnp
from jax import lax
from jax.experimental import pallas as pl
from jax.experimental.pallas import tpu as pltpu
from jax.experimental.pallas import tpu_sc as plsc

N_NODE = 10000
D = 256
E_TOTAL = 160000
NC, NS = 2, 16

NP = 10240
NAGG = 10112
EPT = 10240
E_PAD = NS * EPT
CHUNK = 128
BLK1 = 1024
BLK2 = 1000


def _project_body(x_ref, w_ref, b_ref, o_ref):
    jj = pl.program_id(0)
    is_fac = (jj < NP // BLK1).astype(jnp.float32)
    acc = jnp.dot(x_ref[...], w_ref[...], preferred_element_type=jnp.float32)
    p = (acc + is_fac * b_ref[...]).astype(jnp.bfloat16)
    o_ref[0] = p[:, 0:128]
    o_ref[1] = p[:, 128:256]


def _project(x_pad, w_msg, b2):
    grid = (2 * NP // BLK1,)
    return pl.pallas_call(
        _project_body,
        grid=grid,
        in_specs=[
            pl.BlockSpec((BLK1, D), lambda j: (j, 0)),
            pl.BlockSpec((D, D), lambda j: (j // (NP // BLK1), 0)),
            pl.BlockSpec((1, D), lambda j: (0, 0)),
        ],
        out_specs=pl.BlockSpec((2, BLK1, 128), lambda j: (0, j, 0)),
        out_shape=jax.ShapeDtypeStruct((2, 2 * NP, 128), jnp.bfloat16),
    )(x_pad, w_msg, b2)


def _edge_sc_body(htab, snd, rcv, out, agg, svtab, rall, sall,
                  gf0, gf1, gv0, gv1, rs,
                  pf0, pf1, pv0, pv1, mg0,
                  sgf0, sgf1, sgv0, sgv1):
    c = lax.axis_index("c")
    s = lax.axis_index("s")
    GF, GV = (gf0, gf1), (gv0, gv1)
    PF, PV = (pf0, pf1), (pv0, pv1)
    SGF, SGV = (sgf0, sgf1), (sgv0, sgv1)
    EPT2 = EPT // 2
    NCHH = EPT2 // CHUNK
    TROWS = NAGG // NS

    offf = c * (2 * NP)
    offv = c * (2 * NP) + NP

    def _zrow(i, _):
        for j in range(4):
            mg0[i, pl.ds(j * 32, 32)] = jnp.zeros((32,), jnp.bfloat16)
        return _
    lax.fori_loop(0, CHUNK, _zrow, None)
    for k in range(TROWS // CHUNK):
        pltpu.sync_copy(mg0, agg.at[pl.ds(s * TROWS + k * CHUNK, CHUNK)])
    rem = TROWS % CHUNK
    if rem:
        pltpu.sync_copy(mg0.at[pl.ds(0, rem)],
                        agg.at[pl.ds(s * TROWS + (TROWS // CHUNK) * CHUNK, rem)])

    @pl.when(s < NS - 1)
    def _():
        pltpu.sync_copy(htab.at[pl.ds(offv + s * 640, 640)],
                        svtab.at[pl.ds(s * 640, 640)])

    @pl.when(s == NS - 1)
    def _():
        pltpu.sync_copy(htab.at[pl.ds(offv + (NS - 1) * 640, 400)],
                        svtab.at[pl.ds((NS - 1) * 640, 400)])
    plsc.subcore_barrier()

    def prep_gidx(k, slot):
        base = k * CHUNK
        for j in range(CHUNK // 32):
            ra, rb = plsc.unpack(rall[pl.ds(base + j * 32, 32)],
                                 format=plsc.PackFormat.INTERLEAVED,
                                 preferred_element_type=jnp.int32)
            GF[slot][pl.ds(j * 32, 16)] = ra + offf
            GF[slot][pl.ds(j * 32 + 16, 16)] = rb + offf
            sa, sb = plsc.unpack(sall[pl.ds(base + j * 32, 32)],
                                 format=plsc.PackFormat.INTERLEAVED,
                                 preferred_element_type=jnp.int32)
            GV[slot][pl.ds(j * 32, 16)] = sa
            GV[slot][pl.ds(j * 32 + 16, 16)] = sb

    def issue_gather(k, slot):
        prep_gidx(k, slot)
        pltpu.async_copy(htab.at[GF[slot]], PF[slot], SGF[slot])
        pltpu.async_copy(svtab.at[GV[slot]], PV[slot], SGV[slot])

    def wait_gather(slot):
        pltpu.make_async_copy(htab.at[GF[slot]], PF[slot], SGF[slot]).wait()
        pltpu.make_async_copy(svtab.at[GV[slot]], PV[slot], SGV[slot]).wait()

    def compute(slot):
        def _row(i, _2):
            a = [PF[slot][i, pl.ds(j * 32, 32)] for j in range(4)]
            b = [PV[slot][i, pl.ds(j * 32, 32)] for j in range(4)]
            m = [jnp.maximum(a[j] + b[j], jnp.bfloat16(0.0)) for j in range(4)]
            for j in range(4):
                mg0[i, pl.ds(j * 32, 32)] = m[j]
            return _2
        lax.fori_loop(0, CHUNK, _row, None, unroll=4)

    def prep_sidx(k):
        base = k * CHUNK
        for j in range(CHUNK // 32):
            ra, rb = plsc.unpack(rall[pl.ds(base + j * 32, 32)],
                                 format=plsc.PackFormat.INTERLEAVED,
                                 preferred_element_type=jnp.int32)
            rs[pl.ds(j * 32, 16)] = ra
            rs[pl.ds(j * 32 + 16, 16)] = rb

    for hh in (0, 1):
        pltpu.sync_copy(rcv.at[pl.ds(s * EPT + hh * EPT2, EPT2)], rall)
        pltpu.sync_copy(snd.at[pl.ds(s * EPT + hh * EPT2, EPT2)], sall)

        issue_gather(0, 0)
        issue_gather(1, 1)
        for slot in (0, 1):
            wait_gather(slot)
            compute(slot)
            prep_sidx(slot)
            pltpu.sync_copy(mg0, agg.at[rs], add=True)
            issue_gather(slot + 2, slot)

        def _pair(p, _):
            for slot in (0, 1):
                k = 2 * p + slot
                wait_gather(slot)
                compute(slot)
                prep_sidx(k)
                pltpu.sync_copy(mg0, agg.at[rs], add=True)

                @pl.when(k + 2 < NCHH)
                def _():
                    issue_gather(k + 2, slot)
            return _
        lax.fori_loop(1, NCHH // 2, _pair, None)
    plsc.subcore_barrier()

    pltpu.sync_copy(agg.at[pl.ds(s * TROWS, TROWS)],
                    out.at[c, pl.ds(s * TROWS, TROWS)])


@functools.cache
def _edge_sc():
    mesh = plsc.VectorSubcoreMesh(
        core_axis_name="c", subcore_axis_name="s", num_cores=NC, num_subcores=NS)
    return pl.kernel(
        _edge_sc_body,
        out_type=jax.ShapeDtypeStruct((NC, NAGG, 128), jnp.bfloat16),
        mesh=mesh,
        scratch_types=(
            [pltpu.VMEM_SHARED((NAGG, 128), jnp.bfloat16)]
            + [pltpu.VMEM_SHARED((N_NODE, 128), jnp.bfloat16)]
            + [pltpu.VMEM((EPT // 2,), jnp.int16)] * 2
            + [pltpu.VMEM((CHUNK,), jnp.int32)] * 5
            + [pltpu.VMEM((CHUNK, 128), jnp.bfloat16)] * 5
            + [pltpu.SemaphoreType.DMA] * 4
        ),
        compiler_params=pltpu.CompilerParams(
            use_tc_tiling_on_sc=False, needs_layout_passes=False),
    )


def _combine_body(f_ref, a_ref, w_ref, b_ref, o_ref):
    acc = jnp.dot(f_ref[...], w_ref[0:256, :], preferred_element_type=jnp.float32)
    acc += jnp.dot(a_ref[0].astype(jnp.float32), w_ref[256:384, :],
                   preferred_element_type=jnp.float32)
    acc += jnp.dot(a_ref[1].astype(jnp.float32), w_ref[384:512, :],
                   preferred_element_type=jnp.float32)
    o_ref[...] = jnp.maximum(acc + b_ref[...], 0.0)


def _combine(factors, hagg, w_comb, b2):
    grid = (N_NODE // BLK2,)
    return pl.pallas_call(
        _combine_body,
        grid=grid,
        in_specs=[
            pl.BlockSpec((BLK2, D), lambda i: (i, 0)),
            pl.BlockSpec((NC, BLK2, 128), lambda i: (0, i, 0)),
            pl.BlockSpec((2 * D, D), lambda i: (0, 0)),
            pl.BlockSpec((1, D), lambda i: (0, 0)),
        ],
        out_specs=pl.BlockSpec((BLK2, D), lambda i: (i, 0)),
        out_shape=jax.ShapeDtypeStruct((N_NODE, D), jnp.float32),
    )(factors, hagg, w_comb, b2)


def kernel(variables, factors, senders, receivers, W_msg, b_msg, W_comb, b_comb):
    x_pad = jnp.zeros((2 * NP, D), jnp.float32)
    x_pad = x_pad.at[0:N_NODE].set(factors)
    x_pad = x_pad.at[NP:NP + N_NODE].set(variables)

    p_all = _project(x_pad, W_msg, b_msg.reshape(1, D))
    htab = p_all.reshape(4 * NP, 128)

    snd_pad = jnp.concatenate(
        [senders, jnp.zeros((E_PAD - E_TOTAL,), jnp.int32)]).astype(jnp.int16)
    rcv_pad = jnp.concatenate(
        [receivers, jnp.full((E_PAD - E_TOTAL,), N_NODE, jnp.int32)]).astype(jnp.int16)

    hagg = _edge_sc()(htab, snd_pad, rcv_pad)

    out = _combine(factors, hagg, W_comb, b_comb.reshape(1, D))
    return out


# speedup vs baseline: 5.0963x; 1.0667x over previous
import functools

import jax
import jax.numpy as jnp
from jax import lax
from jax.experimental import pallas as pl
from jax.experimental.pallas import tpu as pltpu
from jax.experimental.pallas import tpu_sc as plsc

N_NODE = 10000
D = 256
E_TOTAL = 160000
NC, NS = 2, 16

NP = 10240
NAGG = 10112
EPT = 10240
E_PAD = NS * EPT
CHUNK = 128
BLK1 = 1024
BLK2 = 1000


def _project_body(x_ref, w_ref, b_ref, o_ref):
    jj = pl.program_id(0)
    is_fac = (jj < NP // BLK1).astype(jnp.float32)
    acc = jnp.dot(x_ref[...], w_ref[...], preferred_element_type=jnp.float32)
    p = (acc + is_fac * b_ref[...]).astype(jnp.bfloat16)
    o_ref[0] = p[:, 0:128]
    o_ref[1] = p[:, 128:256]


def _project(x_pad, w_msg, b2):
    grid = (2 * NP // BLK1,)
    return pl.pallas_call(
        _project_body,
        grid=grid,
        in_specs=[
            pl.BlockSpec((BLK1, D), lambda j: (j, 0)),
            pl.BlockSpec((D, D), lambda j: (j // (NP // BLK1), 0)),
            pl.BlockSpec((1, D), lambda j: (0, 0)),
        ],
        out_specs=pl.BlockSpec((2, BLK1, 128), lambda j: (0, j, 0)),
        out_shape=jax.ShapeDtypeStruct((2, 2 * NP, 128), jnp.bfloat16),
    )(x_pad, w_msg, b2)


def _edge_sc_body(htab, snd, rcv, out, agg, rall, sall,
                  gf0, gf1, gv0, gv1, rs0, rs1,
                  pf0, pf1, pv0, pv1, mg0, mg1,
                  sgf0, sgf1, sgv0, sgv1, ssc0, ssc1):
    c = lax.axis_index("c")
    s = lax.axis_index("s")
    GF, GV, RS = (gf0, gf1), (gv0, gv1), (rs0, rs1)
    PF, PV, MG = (pf0, pf1), (pv0, pv1), (mg0, mg1)
    SGF, SGV, SSC = (sgf0, sgf1), (sgv0, sgv1), (ssc0, ssc1)
    NCH = EPT // CHUNK
    TROWS = NAGG // NS

    pltpu.sync_copy(rcv.at[pl.ds(s * EPT, EPT)], rall)
    pltpu.sync_copy(snd.at[pl.ds(s * EPT, EPT)], sall)

    offf = c * (2 * NP)
    offv = c * (2 * NP) + NP

    def _zrow(i, _):
        for j in range(4):
            mg0[i, pl.ds(j * 32, 32)] = jnp.zeros((32,), jnp.bfloat16)
        return _
    lax.fori_loop(0, CHUNK, _zrow, None)
    for k in range(TROWS // CHUNK):
        pltpu.sync_copy(mg0, agg.at[pl.ds(s * TROWS + k * CHUNK, CHUNK)])
    rem = TROWS % CHUNK
    if rem:
        pltpu.sync_copy(mg0.at[pl.ds(0, rem)],
                        agg.at[pl.ds(s * TROWS + (TROWS // CHUNK) * CHUNK, rem)])
    plsc.subcore_barrier()

    def prep_gidx(k, slot):
        base = k * CHUNK
        for j in range(CHUNK // 16):
            sl = pl.ds(j * 16, 16)
            GF[slot][sl] = rall[pl.ds(base + j * 16, 16)] + offf
            GV[slot][sl] = sall[pl.ds(base + j * 16, 16)] + offv

    def issue_gather(k, slot):
        prep_gidx(k, slot)
        pltpu.async_copy(htab.at[GF[slot]], PF[slot], SGF[slot])
        pltpu.async_copy(htab.at[GV[slot]], PV[slot], SGV[slot])

    def wait_gather(slot):
        pltpu.make_async_copy(htab.at[GF[slot]], PF[slot], SGF[slot]).wait()
        pltpu.make_async_copy(htab.at[GV[slot]], PV[slot], SGV[slot]).wait()

    def compute(slot):
        def _row(i, _2):
            a = [PF[slot][i, pl.ds(j * 32, 32)] for j in range(4)]
            b = [PV[slot][i, pl.ds(j * 32, 32)] for j in range(4)]
            m = [jnp.maximum(a[j] + b[j], jnp.bfloat16(0.0)) for j in range(4)]
            for j in range(4):
                MG[slot][i, pl.ds(j * 32, 32)] = m[j]
            return _2
        lax.fori_loop(0, CHUNK, _row, None, unroll=4)

    def prep_sidx(k, slot):
        base = k * CHUNK
        for j in range(CHUNK // 16):
            RS[slot][pl.ds(j * 16, 16)] = rall[pl.ds(base + j * 16, 16)]

    def issue_scatter(slot):
        pltpu.async_copy(MG[slot], agg.at[RS[slot]], SSC[slot], add=True)

    def wait_scatter(slot):
        pltpu.make_async_copy(MG[slot], agg.at[RS[slot]], SSC[slot]).wait()

    issue_gather(0, 0)
    issue_gather(1, 1)
    for slot in (0, 1):
        wait_gather(slot)
        compute(slot)
        prep_sidx(slot, slot)
        issue_scatter(slot)
        issue_gather(slot + 2, slot)

    def _pair(p, _):
        for slot in (0, 1):
            k = 2 * p + slot
            wait_gather(slot)
            wait_scatter(slot)
            compute(slot)
            prep_sidx(k, slot)
            issue_scatter(slot)

            @pl.when(k + 2 < NCH)
            def _():
                issue_gather(k + 2, slot)
        return _
    lax.fori_loop(1, NCH // 2, _pair, None)
    wait_scatter(0)
    wait_scatter(1)
    plsc.subcore_barrier()

    pltpu.sync_copy(agg.at[pl.ds(s * TROWS, TROWS)],
                    out.at[c, pl.ds(s * TROWS, TROWS)])


@functools.cache
def _edge_sc():
    mesh = plsc.VectorSubcoreMesh(
        core_axis_name="c", subcore_axis_name="s", num_cores=NC, num_subcores=NS)
    return pl.kernel(
        _edge_sc_body,
        out_type=jax.ShapeDtypeStruct((NC, NAGG, 128), jnp.bfloat16),
        mesh=mesh,
        scratch_types=(
            [pltpu.VMEM_SHARED((NAGG, 128), jnp.bfloat16)]
            + [pltpu.VMEM((EPT,), jnp.int32)] * 2
            + [pltpu.VMEM((CHUNK,), jnp.int32)] * 6
            + [pltpu.VMEM((CHUNK, 128), jnp.bfloat16)] * 6
            + [pltpu.SemaphoreType.DMA] * 6
        ),
        compiler_params=pltpu.CompilerParams(use_tc_tiling_on_sc=False),
    )


def _combine_body(f_ref, a_ref, w_ref, b_ref, o_ref):
    acc = jnp.dot(f_ref[...], w_ref[0:256, :], preferred_element_type=jnp.float32)
    acc += jnp.dot(a_ref[0].astype(jnp.float32), w_ref[256:384, :],
                   preferred_element_type=jnp.float32)
    acc += jnp.dot(a_ref[1].astype(jnp.float32), w_ref[384:512, :],
                   preferred_element_type=jnp.float32)
    o_ref[...] = jnp.maximum(acc + b_ref[...], 0.0)


def _combine(factors, hagg, w_comb, b2):
    grid = (N_NODE // BLK2,)
    return pl.pallas_call(
        _combine_body,
        grid=grid,
        in_specs=[
            pl.BlockSpec((BLK2, D), lambda i: (i, 0)),
            pl.BlockSpec((NC, BLK2, 128), lambda i: (0, i, 0)),
            pl.BlockSpec((2 * D, D), lambda i: (0, 0)),
            pl.BlockSpec((1, D), lambda i: (0, 0)),
        ],
        out_specs=pl.BlockSpec((BLK2, D), lambda i: (i, 0)),
        out_shape=jax.ShapeDtypeStruct((N_NODE, D), jnp.float32),
    )(factors, hagg, w_comb, b2)


def kernel(variables, factors, senders, receivers, W_msg, b_msg, W_comb, b_comb):
    x_pad = jnp.zeros((2 * NP, D), jnp.float32)
    x_pad = x_pad.at[0:N_NODE].set(factors)
    x_pad = x_pad.at[NP:NP + N_NODE].set(variables)

    p_all = _project(x_pad, W_msg, b_msg.reshape(1, D))
    htab = p_all.reshape(4 * NP, 128)

    snd_pad = jnp.concatenate([senders, jnp.zeros((E_PAD - E_TOTAL,), jnp.int32)])
    rcv_pad = jnp.concatenate(
        [receivers, jnp.full((E_PAD - E_TOTAL,), N_NODE, jnp.int32)])

    hagg = _edge_sc()(htab, snd_pad, rcv_pad)

    out = _combine(factors, hagg, W_comb, b_comb.reshape(1, D))
    return out
